# baseline (speedup 1.0000x reference)
import functools

import jax
import jax.numpy as jnp
from jax import lax
from jax.experimental import pallas as pl
from jax.experimental.pallas import tpu as pltpu
from jax.experimental.pallas import tpu_sc as plsc

_L = 16
_N = 64

_mesh = plsc.VectorSubcoreMesh(core_axis_name="c", subcore_axis_name="s")


@functools.partial(
    pl.kernel,
    mesh=_mesh,
    out_type=jax.ShapeDtypeStruct((_N,), jnp.float32),
    scratch_types=[pltpu.VMEM((_N,), jnp.float32)],
)
def _router_sigmoid(prob_hbm, out_hbm, buf):
    wid = lax.axis_index("s") * 2 + lax.axis_index("c")

    @pl.when(wid == 0)
    def _():
        pltpu.sync_copy(prob_hbm, buf)
        for i in range(_N // _L):
            sl = pl.ds(i * _L, _L)
            x = buf[sl]
            buf[sl] = 1.0 / (1.0 + jnp.exp(-x))
        pltpu.sync_copy(buf, out_hbm)


def kernel(prob):
    return _router_sigmoid(prob)


# speedup vs baseline: 1.0645x; 1.0645x over previous
import functools

import jax
import jax.numpy as jnp
from jax import lax
from jax.experimental import pallas as pl
from jax.experimental.pallas import tpu as pltpu
from jax.experimental.pallas import tpu_sc as plsc

_L = 16
_N = 64

_mesh = plsc.VectorSubcoreMesh(
    core_axis_name="c", subcore_axis_name="s", num_cores=1
)


@functools.partial(
    pl.kernel,
    mesh=_mesh,
    out_type=jax.ShapeDtypeStruct((_N,), jnp.float32),
    scratch_types=[pltpu.VMEM((_N,), jnp.float32)],
)
def _router_sigmoid(prob_hbm, out_hbm, buf):
    wid = lax.axis_index("s") * 2 + lax.axis_index("c")

    @pl.when(wid == 0)
    def _():
        pltpu.sync_copy(prob_hbm, buf)
        for i in range(_N // _L):
            sl = pl.ds(i * _L, _L)
            x = buf[sl]
            buf[sl] = 1.0 / (1.0 + jnp.exp(-x))
        pltpu.sync_copy(buf, out_hbm)


def kernel(prob):
    return _router_sigmoid(prob)


# speedup vs baseline: 1.0748x; 1.0096x over previous
import functools

import jax
import jax.numpy as jnp
from jax import lax
from jax.experimental import pallas as pl
from jax.experimental.pallas import tpu as pltpu
from jax.experimental.pallas import tpu_sc as plsc

_L = 16
_N = 64

_mesh = plsc.VectorSubcoreMesh(
    core_axis_name="c", subcore_axis_name="s", num_cores=1, num_subcores=1
)


@functools.partial(
    pl.kernel,
    mesh=_mesh,
    out_type=jax.ShapeDtypeStruct((_N,), jnp.float32),
    scratch_types=[pltpu.VMEM((_N,), jnp.float32)],
)
def _router_sigmoid(prob_hbm, out_hbm, buf):
    wid = lax.axis_index("s") * 2 + lax.axis_index("c")

    @pl.when(wid == 0)
    def _():
        pltpu.sync_copy(prob_hbm, buf)
        for i in range(_N // _L):
            sl = pl.ds(i * _L, _L)
            x = buf[sl]
            buf[sl] = 1.0 / (1.0 + jnp.exp(-x))
        pltpu.sync_copy(buf, out_hbm)


def kernel(prob):
    return _router_sigmoid(prob)


# speedup vs baseline: 1.0867x; 1.0111x over previous
import functools

import jax
import jax.numpy as jnp
from jax import lax
from jax.experimental import pallas as pl
from jax.experimental.pallas import tpu as pltpu
from jax.experimental.pallas import tpu_sc as plsc

_L = 16
_N = 64

_mesh = plsc.VectorSubcoreMesh(
    core_axis_name="c", subcore_axis_name="s", num_cores=1, num_subcores=4
)


@functools.partial(
    pl.kernel,
    mesh=_mesh,
    out_type=jax.ShapeDtypeStruct((_N,), jnp.float32),
    scratch_types=[pltpu.VMEM((_L,), jnp.float32)],
)
def _router_sigmoid(prob_hbm, out_hbm, buf):
    wid = lax.axis_index("s")
    base = wid * _L
    pltpu.sync_copy(prob_hbm.at[pl.ds(base, _L)], buf)
    x = buf[...]
    buf[...] = 1.0 / (1.0 + jnp.exp(-x))
    pltpu.sync_copy(buf, out_hbm.at[pl.ds(base, _L)])


def kernel(prob):
    return _router_sigmoid(prob)
